# Initial kernel scaffold; baseline (speedup 1.0000x reference)
import functools

import jax
import jax.numpy as jnp
from jax import lax
from jax.experimental import pallas as pl
from jax.experimental.pallas import tpu as pltpu
from jax.experimental.pallas import tpu_sc as plsc

_B = 16384
_NFEAT = 39
_NCONT = 13
_NCAT = 26
_VOCAB = 100000
_EDIM = 16
_OUTD = _NCONT + _NCAT * _EDIM

_NC, _NS = 2, 16
_NW = _NC * _NS
_BPW = _B // _NW
_BSUB = 64
_NITER = _BPW // _BSUB

_mesh = plsc.VectorSubcoreMesh(core_axis_name="c", subcore_axis_name="s")


@functools.partial(
    pl.kernel,
    out_type=jax.ShapeDtypeStruct((_B, _OUTD), jnp.float32),
    mesh=_mesh,
    scratch_types=[
        pltpu.VMEM((_BSUB, _NFEAT), jnp.int32),
        pltpu.VMEM((_NCAT * _BSUB,), jnp.int32),
        pltpu.VMEM((_NCAT, _BSUB, _EDIM), jnp.float32),
        pltpu.VMEM((_BSUB, _OUTD), jnp.float32),
        pltpu.SemaphoreType.DMA,
    ],
)
def _emb_kernel(x_hbm, tbl_hbm, out_hbm, xbuf, idxbuf, gbuf, chunk, sem):
    wid = lax.axis_index("s") * _NC + lax.axis_index("c")
    base = wid * _BPW
    iota = lax.iota(jnp.int32, 16)

    def body(it, carry):
        r0 = pl.multiple_of(base + it * _BSUB, _BSUB)
        pltpu.sync_copy(x_hbm.at[pl.ds(r0, _BSUB)], xbuf)

        for j in range(_NCAT):
            col = jnp.full((16,), _NCONT + j, jnp.int32)
            for k in range(_BSUB // 16):
                rows = k * 16 + iota
                v = plsc.load_gather(xbuf, [rows, col])
                idxbuf[pl.ds(j * _BSUB + k * 16, 16)] = v + j * _VOCAB

        copies = [
            pltpu.make_async_copy(
                tbl_hbm.at[idxbuf.at[pl.ds(j * _BSUB, _BSUB)]],
                gbuf.at[j],
                sem,
            )
            for j in range(_NCAT)
        ]
        for cpy in copies:
            cpy.start()

        for c in range(_NCONT):
            col = jnp.full((16,), c, jnp.int32)
            for k in range(_BSUB // 16):
                rows = k * 16 + iota
                v = plsc.load_gather(xbuf, [rows, col])
                plsc.store_scatter(chunk, [rows, col], v.astype(jnp.float32))

        for cpy in copies:
            cpy.wait()
        for j in range(_NCAT):
            pltpu.sync_copy(
                gbuf.at[j], chunk.at[:, pl.ds(_NCONT + j * _EDIM, _EDIM)]
            )

        pltpu.sync_copy(chunk, out_hbm.at[pl.ds(r0, _BSUB)])
        return carry

    lax.fori_loop(0, _NITER, body, 0)


def kernel(x, tables):
    tbl = tables.reshape(_NCAT * _VOCAB, _EDIM)
    return _emb_kernel(x, tbl)


# speedup vs baseline: 1.2145x; 1.2145x over previous
import functools

import jax
import jax.numpy as jnp
from jax import lax
from jax.experimental import pallas as pl
from jax.experimental.pallas import tpu as pltpu
from jax.experimental.pallas import tpu_sc as plsc

_B = 16384
_NFEAT = 39
_NCONT = 13
_NCAT = 26
_VOCAB = 100000
_EDIM = 16
_OUTD = _NCONT + _NCAT * _EDIM

_NC, _NS = 2, 16
_NW = _NC * _NS
_BPW = _B // _NW
_BSUB = 64
_NITER = _BPW // _BSUB

_mesh = plsc.VectorSubcoreMesh(core_axis_name="c", subcore_axis_name="s")


@functools.partial(
    pl.kernel,
    out_type=jax.ShapeDtypeStruct((_B * _OUTD,), jnp.float32),
    mesh=_mesh,
    scratch_types=[
        pltpu.VMEM((_BSUB * _NFEAT,), jnp.int32),
        pltpu.VMEM((_NCAT * _BSUB,), jnp.int32),
        pltpu.VMEM((_NCAT, _BSUB, _EDIM), jnp.float32),
        pltpu.VMEM((_BSUB * _OUTD,), jnp.float32),
        pltpu.SemaphoreType.DMA,
    ],
    compiler_params=pltpu.CompilerParams(
        use_tc_tiling_on_sc=False, needs_layout_passes=False
    ),
)
def _emb_kernel(x_hbm, tbl_hbm, out_hbm, xbuf, idxbuf, gbuf, chunk, sem):
    wid = lax.axis_index("s") * _NC + lax.axis_index("c")
    base = wid * _BPW
    iota = lax.iota(jnp.int32, 16)

    def body(it, carry):
        r0 = pl.multiple_of(base + it * _BSUB, _BSUB)
        pltpu.sync_copy(x_hbm.at[pl.ds(r0 * _NFEAT, _BSUB * _NFEAT)], xbuf)

        for j in range(_NCAT):
            for k in range(_BSUB // 16):
                src = (k * 16 * _NFEAT + _NCONT + j) + iota * _NFEAT
                v = plsc.load_gather(xbuf, [src])
                idxbuf[pl.ds(j * _BSUB + k * 16, 16)] = v + j * _VOCAB

        copies = [
            pltpu.make_async_copy(
                tbl_hbm.at[idxbuf.at[pl.ds(j * _BSUB, _BSUB)]],
                gbuf.at[j],
                sem,
            )
            for j in range(_NCAT)
        ]
        for cpy in copies:
            cpy.start()

        for c in range(_NCONT):
            for k in range(_BSUB // 16):
                src = (k * 16 * _NFEAT + c) + iota * _NFEAT
                dst = (k * 16 * _OUTD + c) + iota * _OUTD
                v = plsc.load_gather(xbuf, [src])
                plsc.store_scatter(chunk, [dst], v.astype(jnp.float32))

        for cpy in copies:
            cpy.wait()

        def place(i, carry2):
            dbase = i * _OUTD + _NCONT
            for j in range(_NCAT):
                v = gbuf[j, i]
                plsc.store_scatter(chunk, [dbase + j * _EDIM + iota], v)
            return carry2

        lax.fori_loop(0, _BSUB, place, 0)

        pltpu.sync_copy(
            chunk,
            out_hbm.at[pl.ds(pl.multiple_of(r0 * _OUTD, _BSUB * _OUTD),
                             _BSUB * _OUTD)],
        )
        return carry

    lax.fori_loop(0, _NITER, body, 0)


def kernel(x, tables):
    tbl = tables.reshape(_NCAT * _VOCAB, _EDIM)
    out = _emb_kernel(x.reshape(_B * _NFEAT), tbl)
    return out.reshape(_B, _OUTD)


# speedup vs baseline: 1.2159x; 1.0012x over previous
import functools

import jax
import jax.numpy as jnp
from jax import lax
from jax.experimental import pallas as pl
from jax.experimental.pallas import tpu as pltpu
from jax.experimental.pallas import tpu_sc as plsc

_B = 16384
_NFEAT = 39
_NCONT = 13
_NCAT = 26
_VOCAB = 100000
_EDIM = 16
_OUTD = _NCONT + _NCAT * _EDIM

_NC, _NS = 2, 16
_NW = _NC * _NS
_BPW = _B // _NW
_BSUB = 64
_NITER = _BPW // _BSUB

_mesh = plsc.VectorSubcoreMesh(core_axis_name="c", subcore_axis_name="s")


@functools.partial(
    pl.kernel,
    out_type=jax.ShapeDtypeStruct((_B * _OUTD,), jnp.float32),
    mesh=_mesh,
    scratch_types=[
        pltpu.VMEM((_BSUB * _NFEAT,), jnp.int32),
        pltpu.VMEM((_NCAT * _BSUB,), jnp.int32),
        pltpu.VMEM((_NCAT, _BSUB, _EDIM), jnp.float32),
        pltpu.VMEM((_BSUB * _OUTD,), jnp.float32),
        pltpu.SemaphoreType.DMA,
    ],
    compiler_params=pltpu.CompilerParams(
        use_tc_tiling_on_sc=False, needs_layout_passes=False
    ),
)
def _emb_kernel(x_hbm, tbl_hbm, out_hbm, xbuf, idxbuf, gbuf, chunk, sem):
    wid = lax.axis_index("s") * _NC + lax.axis_index("c")
    base = wid * _BPW
    iota = lax.iota(jnp.int32, 16)

    def body(it, carry):
        r0 = pl.multiple_of(base + it * _BSUB, _BSUB)
        pltpu.sync_copy(x_hbm.at[pl.ds(r0 * _NFEAT, _BSUB * _NFEAT)], xbuf)

        for j in range(_NCAT):
            for k in range(_BSUB // 16):
                src = (k * 16 * _NFEAT + _NCONT + j) + iota * _NFEAT
                v = plsc.load_gather(xbuf, [src])
                idxbuf[pl.ds(j * _BSUB + k * 16, 16)] = v

        copies = [
            pltpu.make_async_copy(
                tbl_hbm.at[j].at[idxbuf.at[pl.ds(j * _BSUB, _BSUB)]],
                gbuf.at[j],
                sem,
            )
            for j in range(_NCAT)
        ]
        for cpy in copies:
            cpy.start()

        for c in range(_NCONT):
            for k in range(_BSUB // 16):
                src = (k * 16 * _NFEAT + c) + iota * _NFEAT
                dst = (k * 16 * _OUTD + c) + iota * _OUTD
                v = plsc.load_gather(xbuf, [src])
                plsc.store_scatter(chunk, [dst], v.astype(jnp.float32))

        for cpy in copies:
            cpy.wait()

        def place(i, carry2):
            dbase = i * _OUTD + _NCONT
            for j in range(_NCAT):
                v = gbuf[j, i]
                plsc.store_scatter(chunk, [dbase + j * _EDIM + iota], v)
            return carry2

        lax.fori_loop(0, _BSUB, place, 0)

        pltpu.sync_copy(
            chunk,
            out_hbm.at[pl.ds(pl.multiple_of(r0 * _OUTD, _BSUB * _OUTD),
                             _BSUB * _OUTD)],
        )
        return carry

    lax.fori_loop(0, _NITER, body, 0)


def kernel(x, tables):
    out = _emb_kernel(x.reshape(_B * _NFEAT), tables)
    return out.reshape(_B, _OUTD)


# speedup vs baseline: 2.9488x; 2.4253x over previous
import functools

import jax
import jax.numpy as jnp
from jax import lax
from jax.experimental import pallas as pl
from jax.experimental.pallas import tpu as pltpu
from jax.experimental.pallas import tpu_sc as plsc

_B = 16384
_NCONT = 13
_NCAT = 26
_VOCAB = 100000
_VFULL = 781 * 128
_TAIL = _VOCAB - _VFULL

_NW = 32
_BH = _B // 2
_W = 2048
_NWIN_U = _VFULL // _W
_WLAST = _VFULL - _NWIN_U * _W
_NLIST = 50
_CAP = 256
_QC = 512

_mesh = plsc.VectorSubcoreMesh(core_axis_name="c", subcore_axis_name="s")


@functools.partial(
    pl.kernel,
    out_type=jax.ShapeDtypeStruct((54, 128, 8, 128), jnp.float32),
    mesh=_mesh,
    scratch_types=[
        pltpu.VMEM((8, _W), jnp.float32),
        pltpu.VMEM((_BH,), jnp.int32),
        pltpu.VMEM((_BH // 128, 8, 128), jnp.float32),
        pltpu.VMEM((_NLIST * _CAP,), jnp.int32),
        pltpu.VMEM((64,), jnp.int32),
        pltpu.VMEM((48,), jnp.int32),
        pltpu.VMEM((_NCAT * 16 * _TAIL,), jnp.float32),
        pltpu.VMEM((_NCONT * _QC,), jnp.float32),
    ],
    compiler_params=pltpu.CompilerParams(
        use_tc_tiling_on_sc=True, needs_layout_passes=False
    ),
)
def _emb_kernel(tbl4, tail, xi, xc, out5, win, idxb, outb, lists, cnts,
                shf, tailb, contb):
    wid = lax.axis_index("s") * 2 + lax.axis_index("c")
    iota = lax.iota(jnp.int32, 16)
    zeros16 = jnp.zeros((16,), jnp.int32)

    pltpu.sync_copy(tail, tailb)

    def serve(vals_idx, pos, msk, j, h, v0, src_win):
        loc = vals_idx - v0
        oc = lax.shift_right_logical(pos, 7)
        ol = lax.bitwise_and(pos, 127)
        for d in range(8):
            if src_win:
                tv = plsc.load_gather(
                    win, [jnp.full((16,), d, jnp.int32), loc], mask=msk)
            else:
                base = (j * 16 + h * 8 + d) * _TAIL
                tv = plsc.load_gather(tailb, [base + loc], mask=msk)
            plsc.store_scatter(outb, [oc, jnp.full((16,), d, jnp.int32), ol],
                               tv, mask=msk)

    def do_emb(su):
        j = su // 4
        h = (su // 2) % 2
        half = su % 2
        pltpu.sync_copy(xi.at[pl.ds(j * _B + half * _BH, _BH)], idxb)

        cnts[pl.ds(0, 16)] = zeros16
        cnts[pl.ds(16, 16)] = zeros16
        cnts[pl.ds(32, 16)] = zeros16
        cnts[pl.ds(48, 16)] = zeros16

        def bloop(k, carry):
            idx = idxb[pl.ds(k * 16, 16)]
            winid = jnp.where(idx >= _VFULL, _NLIST - 1,
                              lax.shift_right_logical(idx, 11))
            packed = lax.bitwise_or(lax.shift_left(k * 16 + iota, 17), idx)
            skey, spay = plsc.sort_key_val(winid, packed)
            shf[pl.ds(0, 16)] = jnp.full((16,), -1, jnp.int32)
            plsc.store_scatter(shf, [1 + iota], skey)
            prev = plsc.load_gather(shf, [iota])
            newrun = (skey != prev).astype(jnp.int32)
            start = plsc.cummax(newrun * iota)
            rank = iota - start
            base = plsc.load_gather(cnts, [skey])
            slot = base + rank
            ok = slot < _CAP
            plsc.store_scatter(lists, [skey * _CAP + jnp.minimum(slot, _CAP - 1)],
                               spay, mask=ok)
            shf[pl.ds(16, 16)] = newrun
            plsc.store_scatter(shf, [jnp.full((16,), 32, jnp.int32)],
                               jnp.ones((16,), jnp.int32))
            islast = plsc.load_gather(shf, [17 + iota]) != 0
            plsc.store_scatter(cnts, [skey], slot + 1, mask=islast)
            return carry

        lax.fori_loop(0, _BH // 16, bloop, 0)

        def serve_list(wlist, j_, h_, v0, src_win):
            cv = plsc.load_gather(cnts, [jnp.full((16,), wlist, jnp.int32)])
            cnt = cv[0]

            def lloop(v, carry):
                packed = lists[pl.ds(wlist * _CAP + v * 16, 16)]
                msk = (v * 16 + iota) < jnp.minimum(cnt, _CAP)
                idx = lax.bitwise_and(packed, 0x1FFFF)
                pos = lax.shift_right_logical(packed, 17)
                serve(idx, pos, msk, j_, h_, v0, src_win)
                return carry

            nv = lax.div(jnp.minimum(cnt, _CAP) + 15, 16)
            lax.fori_loop(0, nv, lloop, 0)

            @pl.when(cnt > _CAP)
            def _():
                def floop(k, carry):
                    idx = idxb[pl.ds(k * 16, 16)]
                    winid = jnp.where(
                        idx >= _VFULL, _NLIST - 1,
                        lax.shift_right_logical(idx, 11))
                    msk = winid == wlist
                    serve(idx, k * 16 + iota, msk, j_, h_, v0, src_win)
                    return carry

                lax.fori_loop(0, _BH // 16, floop, 0)

        def wloop(w, carry):
            v0 = pl.multiple_of(w * _W, _W)
            pltpu.sync_copy(tbl4.at[j, h, :, pl.ds(v0, _W)], win)
            serve_list(w, j, h, v0, True)
            return carry

        lax.fori_loop(0, _NWIN_U, wloop, 0)

        pltpu.sync_copy(tbl4.at[j, h, :, pl.ds(_NWIN_U * _W, _WLAST)],
                        win.at[:, pl.ds(0, _WLAST)])
        serve_list(_NWIN_U, j, h, _NWIN_U * _W, True)

        serve_list(_NLIST - 1, j, h, _VFULL, False)

        s = 2 + 2 * j + h
        pltpu.sync_copy(outb, out5.at[s, pl.ds(half * (_BH // 128), _BH // 128)])

    def do_cont(q):
        pltpu.sync_copy(xc.at[pl.ds(q * _NCONT * _QC, _NCONT * _QC)], contb)
        for s in range(2):
            for c4 in range(_QC // 128):
                for r in range(8):
                    row = s * 8 + r

                    def kloop(k, carry, row=row, c4=c4, r=r):
                        if row < _NCONT:
                            v = contb[pl.ds(row * _QC + c4 * 128 + k * 16, 16)]
                        else:
                            v = jnp.zeros((16,), jnp.float32)
                        outb[c4, r, pl.ds(k * 16, 16)] = v
                        return carry

                    lax.fori_loop(0, 8, kloop, 0)
            pltpu.sync_copy(
                outb.at[pl.ds(0, _QC // 128)],
                out5.at[s, pl.ds(q * (_QC // 128), _QC // 128)])

    n_emb = _NCAT * 2 * 2
    total = n_emb + _B // _QC

    def uloop(u, carry):
        su = u * _NW + wid

        @pl.when(su < n_emb)
        def _():
            do_emb(su)

        @pl.when((su >= n_emb) & (su < total))
        def _():
            do_cont(su - n_emb)

        return carry
    lax.fori_loop(0, (total + _NW - 1) // _NW, uloop, 0)


def kernel(x, tables):
    tbl4 = tables.transpose(0, 2, 1).reshape(_NCAT, 2, 8, _VOCAB)
    tail = tables[:, _VFULL:, :].transpose(0, 2, 1).reshape(-1)
    xi = x[:, _NCONT:].T.reshape(-1)
    xc = (
        x[:, :_NCONT].T.astype(jnp.float32)
        .reshape(_NCONT, _B // _QC, _QC).transpose(1, 0, 2).reshape(-1)
    )
    out5 = _emb_kernel(tbl4, tail, xi, xc)
    o = out5.transpose(0, 2, 1, 3).reshape(432, _B)
    return jnp.concatenate([o[:_NCONT], o[16:]], axis=0).T
